# baseline (speedup 1.0000x reference)
import functools

import jax
import jax.numpy as jnp
from jax.experimental import pallas as pl
from jax.experimental.pallas import tpu as pltpu


def _fused_kernel(
    x_ref,
    adj_a,
    adj_b,
    wfc_ref,
    bfc_ref,
    wg1_ref,
    bg1_ref,
    wg2_ref,
    bg2_ref,
    out_ref,
    h1_ref,
    h2_ref,
    *,
    nblk,
    bm,
):
    i = pl.program_id(0)
    half = bm // 2

    @pl.when(i == 0)
    def _():
        h = jnp.dot(x_ref[...], wfc_ref[...], preferred_element_type=jnp.float32)
        h = jnp.maximum(h + bfc_ref[...], 0.0)
        h1_ref[...] = (
            jnp.dot(h, wg1_ref[...], preferred_element_type=jnp.float32)
            + bg1_ref[...]
        )

    @pl.when((i >= 1) & (i <= nblk))
    def _():
        base = (i - 1) * bm
        for k, a_ref in enumerate((adj_a, adj_b)):
            t = jnp.dot(a_ref[...], h1_ref[...], preferred_element_type=jnp.float32)
            t = jnp.maximum(t, 0.0)
            h2_ref[pl.ds(base + k * half, half), :] = (
                jnp.dot(t, wg2_ref[...], preferred_element_type=jnp.float32)
                + bg2_ref[...]
            )

    @pl.when(i > nblk)
    def _():
        for k, a_ref in enumerate((adj_a, adj_b)):
            t = jnp.dot(a_ref[...], h2_ref[...], preferred_element_type=jnp.float32)
            out_ref[k * half : (k + 1) * half, :] = jnp.maximum(t, 0.0)


def kernel(x, adj, W_fc, b_fc, W_g1, b_g1, W_g2, b_g2):
    n, in_ft = x.shape
    h1w = W_g1.shape[1]
    outw = W_g2.shape[1]
    b_fc2 = b_fc.reshape(1, -1)
    b_g12 = b_g1.reshape(1, -1)
    b_g22 = b_g2.reshape(1, -1)

    bm = n
    for cand in (400, 256, 200, 128, 80, 40, 16, 8):
        if n % cand == 0:
            bm = cand
            break
    nblk = n // bm
    half = bm // 2

    full = lambda shape: pl.BlockSpec(shape, lambda i: (0, 0))

    def halfmap(k):
        return lambda i: (2 * ((jnp.maximum(i, 1) - 1) % nblk) + k, 0)

    out = pl.pallas_call(
        functools.partial(_fused_kernel, nblk=nblk, bm=bm),
        grid=(2 * nblk + 1,),
        in_specs=[
            full((n, in_ft)),
            pl.BlockSpec((half, n), halfmap(0)),
            pl.BlockSpec((half, n), halfmap(1)),
            full(W_fc.shape),
            full(b_fc2.shape),
            full(W_g1.shape),
            full(b_g12.shape),
            full(W_g2.shape),
            full(b_g22.shape),
        ],
        out_specs=pl.BlockSpec(
            (bm, outw), lambda i: (jnp.maximum(i - (nblk + 1), 0), 0)
        ),
        out_shape=jax.ShapeDtypeStruct((n, outw), jnp.float32),
        scratch_shapes=[
            pltpu.VMEM((n, h1w), jnp.float32),
            pltpu.VMEM((n, outw), jnp.float32),
        ],
        compiler_params=pltpu.CompilerParams(
            vmem_limit_bytes=64 * 1024 * 1024,
        ),
    )(x, adj, adj, W_fc, b_fc2, W_g1, b_g12, W_g2, b_g22)
    return out


# speedup vs baseline: 1.0410x; 1.0410x over previous
import functools

import jax
import jax.numpy as jnp
from jax.experimental import pallas as pl
from jax.experimental.pallas import tpu as pltpu

_CH = 200
_NSLOT = 4
_DEPTH = 3


def _manual_kernel(
    x_ref,
    adj_ref,
    wfc_ref,
    bfc_ref,
    wg1_ref,
    bg1_ref,
    wg2_ref,
    bg2_ref,
    out_ref,
    h1_ref,
    h2_ref,
    bufs_ref,
    sems,
    *,
    n,
    nch,
):
    total = 2 * nch

    def start_copy(c):
        slot = jax.lax.rem(c, _NSLOT)
        row = jax.lax.rem(c, nch) * _CH
        pltpu.make_async_copy(
            adj_ref.at[pl.ds(row, _CH), :],
            bufs_ref.at[slot],
            sems.at[slot],
        ).start()

    def wait_copy(c):
        slot = jax.lax.rem(c, _NSLOT)
        pltpu.make_async_copy(
            adj_ref.at[pl.ds(0, _CH), :],
            bufs_ref.at[slot],
            sems.at[slot],
        ).wait()

    for c in range(_DEPTH):
        start_copy(c)

    h = jnp.dot(x_ref[...], wfc_ref[...], preferred_element_type=jnp.float32)
    h = jnp.maximum(h + bfc_ref[...], 0.0)
    h1_ref[...] = (
        jnp.dot(h, wg1_ref[...], preferred_element_type=jnp.float32)
        + bg1_ref[...]
    )

    def body(c, _):
        wait_copy(c)

        @pl.when(c + _DEPTH < total)
        def _():
            start_copy(c + _DEPTH)

        slot = jax.lax.rem(c, _NSLOT)
        a = bufs_ref[slot]
        row = jax.lax.rem(c, nch) * _CH

        @pl.when(c < nch)
        def _():
            t = jnp.dot(a, h1_ref[...], preferred_element_type=jnp.float32)
            t = jnp.maximum(t, 0.0)
            h2_ref[pl.ds(row, _CH), :] = (
                jnp.dot(t, wg2_ref[...], preferred_element_type=jnp.float32)
                + bg2_ref[...]
            )

        @pl.when(c >= nch)
        def _():
            t = jnp.dot(a, h2_ref[...], preferred_element_type=jnp.float32)
            out_ref[pl.ds(row, _CH), :] = jnp.maximum(t, 0.0)

        return _

    jax.lax.fori_loop(0, total, body, None)


def kernel(x, adj, W_fc, b_fc, W_g1, b_g1, W_g2, b_g2):
    n, in_ft = x.shape
    h1w = W_g1.shape[1]
    outw = W_g2.shape[1]
    b_fc2 = b_fc.reshape(1, -1)
    b_g12 = b_g1.reshape(1, -1)
    b_g22 = b_g2.reshape(1, -1)

    nch = n // _CH

    vmem = pl.BlockSpec(memory_space=pltpu.MemorySpace.VMEM)
    hbm = pl.BlockSpec(memory_space=pltpu.MemorySpace.HBM)

    out = pl.pallas_call(
        functools.partial(_manual_kernel, n=n, nch=nch),
        in_specs=[vmem, hbm, vmem, vmem, vmem, vmem, vmem, vmem],
        out_specs=vmem,
        out_shape=jax.ShapeDtypeStruct((n, outw), jnp.float32),
        scratch_shapes=[
            pltpu.VMEM((n, h1w), jnp.float32),
            pltpu.VMEM((n, outw), jnp.float32),
            pltpu.VMEM((_NSLOT, _CH, n), jnp.float32),
            pltpu.SemaphoreType.DMA((_NSLOT,)),
        ],
        compiler_params=pltpu.CompilerParams(
            vmem_limit_bytes=64 * 1024 * 1024,
        ),
    )(x, adj, W_fc, b_fc2, W_g1, b_g12, W_g2, b_g22)
    return out


# speedup vs baseline: 1.0507x; 1.0094x over previous
import functools

import jax
import jax.numpy as jnp
from jax.experimental import pallas as pl
from jax.experimental.pallas import tpu as pltpu

_CH = 200
_NSLOT = 4
_DEPTH = 3


def _manual_kernel(
    x_ref,
    adj_ref,
    wfc_ref,
    bfc_ref,
    wg1_ref,
    bg1_ref,
    wg2_ref,
    bg2_ref,
    out_ref,
    h1_ref,
    h2_ref,
    buf0,
    buf1,
    buf2,
    buf3,
    sems,
    *,
    n,
    nch,
):
    total = 2 * nch
    ngroup = total // _NSLOT
    bufs = (buf0, buf1, buf2, buf3)

    def start_copy(c, slot):
        row = jax.lax.rem(c, nch) * _CH
        pltpu.make_async_copy(
            adj_ref.at[pl.ds(row, _CH), :],
            bufs[slot],
            sems.at[slot],
        ).start()

    def wait_copy(slot):
        pltpu.make_async_copy(
            adj_ref.at[pl.ds(0, _CH), :],
            bufs[slot],
            sems.at[slot],
        ).wait()

    for c in range(_DEPTH):
        start_copy(c, c)

    h = jnp.dot(x_ref[...], wfc_ref[...], preferred_element_type=jnp.float32)
    h = jnp.maximum(h + bfc_ref[...], 0.0)
    h1_ref[...] = (
        jnp.dot(h, wg1_ref[...], preferred_element_type=jnp.float32)
        + bg1_ref[...]
    )

    def chunk_body(c, slot):
        wait_copy(slot)

        @pl.when(c + _DEPTH < total)
        def _():
            start_copy(c + _DEPTH, (slot + _DEPTH) % _NSLOT)

        a = bufs[slot][...]
        row = jax.lax.rem(c, nch) * _CH

        @pl.when(c < nch)
        def _():
            t = jnp.dot(a, h1_ref[...], preferred_element_type=jnp.float32)
            t = jnp.maximum(t, 0.0)
            h2_ref[pl.ds(row, _CH), :] = (
                jnp.dot(t, wg2_ref[...], preferred_element_type=jnp.float32)
                + bg2_ref[...]
            )

        @pl.when(c >= nch)
        def _():
            t = jnp.dot(a, h2_ref[...], preferred_element_type=jnp.float32)
            out_ref[pl.ds(row, _CH), :] = jnp.maximum(t, 0.0)

    def body(g, _):
        base = g * _NSLOT
        for k in range(_NSLOT):
            chunk_body(base + k, k)
        return _

    jax.lax.fori_loop(0, ngroup, body, None)
    for k in range(total - ngroup * _NSLOT):
        chunk_body(ngroup * _NSLOT + k, k)


def kernel(x, adj, W_fc, b_fc, W_g1, b_g1, W_g2, b_g2):
    n, in_ft = x.shape
    h1w = W_g1.shape[1]
    outw = W_g2.shape[1]
    b_fc2 = b_fc.reshape(1, -1)
    b_g12 = b_g1.reshape(1, -1)
    b_g22 = b_g2.reshape(1, -1)

    nch = n // _CH

    vmem = pl.BlockSpec(memory_space=pltpu.MemorySpace.VMEM)
    hbm = pl.BlockSpec(memory_space=pltpu.MemorySpace.HBM)

    out = pl.pallas_call(
        functools.partial(_manual_kernel, n=n, nch=nch),
        in_specs=[vmem, hbm, vmem, vmem, vmem, vmem, vmem, vmem],
        out_specs=vmem,
        out_shape=jax.ShapeDtypeStruct((n, outw), jnp.float32),
        scratch_shapes=[
            pltpu.VMEM((n, h1w), jnp.float32),
            pltpu.VMEM((n, outw), jnp.float32),
            pltpu.VMEM((_CH, n), jnp.float32),
            pltpu.VMEM((_CH, n), jnp.float32),
            pltpu.VMEM((_CH, n), jnp.float32),
            pltpu.VMEM((_CH, n), jnp.float32),
            pltpu.SemaphoreType.DMA((_NSLOT,)),
        ],
        compiler_params=pltpu.CompilerParams(
            vmem_limit_bytes=64 * 1024 * 1024,
        ),
    )(x, adj, W_fc, b_fc2, W_g1, b_g12, W_g2, b_g22)
    return out
